# baseline (speedup 1.0000x reference)
import jax
import jax.numpy as jnp
from jax import lax
from jax.experimental import pallas as pl
from jax.experimental.pallas import tpu as pltpu
from jax.experimental.pallas import tpu_sc as plsc

B = 16384
NUM = 13
NCAT = 26
VOCAB = 100000
EDIM = 16
HID = 16
NOUT = 1

VP = 100352
NW = 32
ROWS_PER_W = B // NW
CHUNK = 128
CHUNKS = ROWS_PER_W // CHUNK


def _tc_prep_body(numt_ref, w1_ref, b1_ref, w2_ref, b2_ref, base_ref, wtab_ref):
    w = jnp.dot(w1_ref[...], w2_ref[...], preferred_element_type=jnp.float32)
    c = jnp.dot(b1_ref[...], w2_ref[...], preferred_element_type=jnp.float32) + b2_ref[...]
    wn = w[:NUM, 0].reshape(1, NUM)
    base = jnp.dot(wn, numt_ref[...], preferred_element_type=jnp.float32) + c
    base_ref[...] = base[0, :]
    wtab_ref[...] = w[NUM:, 0].reshape(NCAT, EDIM)


def _tc_prep(numt, W1, b1, W2, b2):
    return pl.pallas_call(
        _tc_prep_body,
        out_shape=[
            jax.ShapeDtypeStruct((B,), jnp.float32),
            jax.ShapeDtypeStruct((NCAT, EDIM), jnp.float32),
        ],
    )(numt, W1, b1, W2, b2)


SPLITS = ((0, 18), (18, 8))


def _make_tc_project_body(start):
    def body(t2_ref, wtab_ref, s_ref):
        j = pl.program_id(0) + start
        mat = t2_ref[...].reshape(EDIM, VOCAB)
        w_row = wtab_ref[pl.ds(j, 1), :]
        vals = jnp.dot(w_row, mat, preferred_element_type=jnp.float32)
        pad = jnp.zeros((1, VP - VOCAB), jnp.float32)
        s_ref[...] = jnp.concatenate([vals, pad], axis=1).reshape(VP)

    return body


def _tc_project(t2, wtab, start, cnt):
    return pl.pallas_call(
        _make_tc_project_body(start),
        grid=(cnt,),
        in_specs=[
            pl.BlockSpec((1, EDIM, VOCAB), lambda j: (j + start, 0, 0)),
            pl.BlockSpec((NCAT, EDIM), lambda j: (0, 0)),
        ],
        out_specs=pl.BlockSpec((VP,), lambda j: (j,)),
        out_shape=jax.ShapeDtypeStruct((cnt * VP,), jnp.float32),
    )(t2, wtab)


def _tree_sum(terms):
    ts = list(terms)
    while len(ts) > 1:
        nxt = [ts[i] + ts[i + 1] for i in range(0, len(ts) - 1, 2)]
        if len(ts) % 2:
            nxt.append(ts[-1])
        ts = nxt
    return ts[0]


def _make_sc_body(start, cnt):
    def body(s_ref, idx_ref, base_ref, out_ref, idx_v, idx_h, sbuf, bbuf, obuf, gsem):
        wid = lax.axis_index("s") * 2 + lax.axis_index("c")
        row0 = wid * ROWS_PER_W

        pltpu.sync_copy(idx_ref.at[:, pl.ds(row0, ROWS_PER_W)], idx_v)
        pltpu.sync_copy(base_ref.at[pl.ds(row0, ROWS_PER_W)], bbuf)

        def add_body(k, _):
            off = k * 16
            for j in range(cnt):
                idx_h[j, pl.ds(off, 16)] = (
                    idx_v[start + j, pl.ds(off, 16)] + j * VP
                )
            return 0

        lax.fori_loop(0, ROWS_PER_W // 16, add_body, 0)

        cps = [
            pltpu.async_copy(
                s_ref.at[idx_h.at[j, pl.ds(c * CHUNK, CHUNK)]],
                sbuf.at[j, pl.ds(c * CHUNK, CHUNK)],
                gsem,
            )
            for j in range(cnt)
            for c in range(CHUNKS)
        ]
        for cp in cps:
            cp.wait()

        def blk_body(k, _):
            off = k * 16
            res = _tree_sum([sbuf[j, pl.ds(off, 16)] for j in range(cnt)])
            obuf[pl.ds(off, 16)] = res + bbuf[pl.ds(off, 16)]
            return 0

        lax.fori_loop(0, ROWS_PER_W // 16, blk_body, 0)
        pltpu.sync_copy(obuf, out_ref.at[pl.ds(row0, ROWS_PER_W)])

    return body


def _sc_gather(s1, idxT, base, start, cnt):
    mesh = plsc.VectorSubcoreMesh(core_axis_name="c", subcore_axis_name="s")
    kfn = pl.kernel(
        _make_sc_body(start, cnt),
        out_type=jax.ShapeDtypeStruct((B,), jnp.float32),
        mesh=mesh,
        compiler_params=pltpu.CompilerParams(use_tc_tiling_on_sc=False),
        scratch_types=[
            pltpu.VMEM((NCAT, ROWS_PER_W), jnp.int32),
            pltpu.VMEM((cnt, ROWS_PER_W), jnp.int32),
            pltpu.VMEM((cnt, ROWS_PER_W), jnp.float32),
            pltpu.VMEM((ROWS_PER_W,), jnp.float32),
            pltpu.VMEM((ROWS_PER_W,), jnp.float32),
            pltpu.SemaphoreType.DMA,
        ],
    )
    return kfn(s1, idxT, base)


@jax.jit
def kernel(numerical, cat_indices, emb_tables, W1, b1, W2, b2):
    base, wtab = _tc_prep(
        jnp.transpose(numerical), W1, b1.reshape(1, HID), W2, b2.reshape(1, NOUT)
    )
    t2 = jnp.transpose(emb_tables, (0, 2, 1))
    idxT = jnp.transpose(cat_indices.astype(jnp.int32))
    s_parts = [_tc_project(t2, wtab, st, cnt) for st, cnt in SPLITS]
    acc = base
    for s_p, (st, cnt) in zip(s_parts, SPLITS):
        acc = _sc_gather(s_p, idxT, acc, st, cnt)
    return acc.reshape(B, NOUT)


# speedup vs baseline: 1.0113x; 1.0113x over previous
import jax
import jax.numpy as jnp
from jax import lax
from jax.experimental import pallas as pl
from jax.experimental.pallas import tpu as pltpu
from jax.experimental.pallas import tpu_sc as plsc

B = 16384
NUM = 13
NCAT = 26
VOCAB = 100000
EDIM = 16
HID = 16
NOUT = 1

VP = 100352
NW = 32
ROWS_PER_W = B // NW
CHUNK = 128
CHUNKS = ROWS_PER_W // CHUNK


def _tc_prep_body(numt_ref, w1_ref, b1_ref, w2_ref, b2_ref, base_ref, wtab_ref):
    w = jnp.dot(w1_ref[...], w2_ref[...], preferred_element_type=jnp.float32)
    c = jnp.dot(b1_ref[...], w2_ref[...], preferred_element_type=jnp.float32) + b2_ref[...]
    wn = w[:NUM, 0].reshape(1, NUM)
    base = jnp.dot(wn, numt_ref[...], preferred_element_type=jnp.float32) + c
    base_ref[...] = base[0, :]
    wtab_ref[...] = w[NUM:, 0].reshape(NCAT, EDIM)


def _tc_prep(numt, W1, b1, W2, b2):
    return pl.pallas_call(
        _tc_prep_body,
        out_shape=[
            jax.ShapeDtypeStruct((B,), jnp.float32),
            jax.ShapeDtypeStruct((NCAT, EDIM), jnp.float32),
        ],
    )(numt, W1, b1, W2, b2)


NH = NCAT // 2


def _make_tc_project_body(half):
    def body(t2_ref, wtab_ref, s_ref):
        j = pl.program_id(0) + half * NH
        mat = t2_ref[...].reshape(EDIM, VOCAB)
        w_row = wtab_ref[pl.ds(j, 1), :]
        vals = jnp.dot(w_row, mat, preferred_element_type=jnp.float32)
        pad = jnp.zeros((1, VP - VOCAB), jnp.float32)
        s_ref[...] = jnp.concatenate([vals, pad], axis=1).reshape(VP)

    return body


def _tc_project(t2, wtab, half):
    return pl.pallas_call(
        _make_tc_project_body(half),
        grid=(NH,),
        in_specs=[
            pl.BlockSpec((1, EDIM, VOCAB), lambda j: (j + half * NH, 0, 0)),
            pl.BlockSpec((NCAT, EDIM), lambda j: (0, 0)),
        ],
        out_specs=pl.BlockSpec((VP,), lambda j: (j,)),
        out_shape=jax.ShapeDtypeStruct((NH * VP,), jnp.float32),
    )(t2, wtab)


def _tree_sum(terms):
    ts = list(terms)
    while len(ts) > 1:
        nxt = [ts[i] + ts[i + 1] for i in range(0, len(ts) - 1, 2)]
        if len(ts) % 2:
            nxt.append(ts[-1])
        ts = nxt
    return ts[0]


def _make_sc_body(half):
    def body(s_ref, idx_ref, base_ref, out_ref, idx_v, idx_h, sbuf, bbuf, obuf, gsem):
        wid = lax.axis_index("s") * 2 + lax.axis_index("c")
        row0 = wid * ROWS_PER_W

        pltpu.sync_copy(idx_ref.at[:, pl.ds(row0, ROWS_PER_W)], idx_v)
        pltpu.sync_copy(base_ref.at[pl.ds(row0, ROWS_PER_W)], bbuf)

        def add_body(k, _):
            off = k * 16
            for j in range(NH):
                idx_h[j, pl.ds(off, 16)] = (
                    idx_v[half * NH + j, pl.ds(off, 16)] + j * VP
                )
            return 0

        lax.fori_loop(0, ROWS_PER_W // 16, add_body, 0)

        cps = [
            pltpu.async_copy(
                s_ref.at[idx_h.at[j, pl.ds(c * CHUNK, CHUNK)]],
                sbuf.at[j, pl.ds(c * CHUNK, CHUNK)],
                gsem,
            )
            for j in range(NH)
            for c in range(CHUNKS)
        ]
        for cp in cps:
            cp.wait()

        def blk_body(k, _):
            off = k * 16
            res = _tree_sum([sbuf[j, pl.ds(off, 16)] for j in range(NH)])
            obuf[pl.ds(off, 16)] = res + bbuf[pl.ds(off, 16)]
            return 0

        lax.fori_loop(0, ROWS_PER_W // 16, blk_body, 0)
        pltpu.sync_copy(obuf, out_ref.at[pl.ds(row0, ROWS_PER_W)])

    return body


def _sc_gather(s1, idxT, base, half):
    mesh = plsc.VectorSubcoreMesh(core_axis_name="c", subcore_axis_name="s")
    kfn = pl.kernel(
        _make_sc_body(half),
        out_type=jax.ShapeDtypeStruct((B,), jnp.float32),
        mesh=mesh,
        compiler_params=pltpu.CompilerParams(use_tc_tiling_on_sc=False),
        scratch_types=[
            pltpu.VMEM((NCAT, ROWS_PER_W), jnp.int32),
            pltpu.VMEM((NH, ROWS_PER_W), jnp.int32),
            pltpu.VMEM((NH, ROWS_PER_W), jnp.float32),
            pltpu.VMEM((ROWS_PER_W,), jnp.float32),
            pltpu.VMEM((ROWS_PER_W,), jnp.float32),
            pltpu.SemaphoreType.DMA,
        ],
    )
    return kfn(s1, idxT, base)


@jax.jit
def kernel(numerical, cat_indices, emb_tables, W1, b1, W2, b2):
    base, wtab = _tc_prep(
        jnp.transpose(numerical), W1, b1.reshape(1, HID), W2, b2.reshape(1, NOUT)
    )
    t2 = jnp.transpose(emb_tables, (0, 2, 1))
    idxT = jnp.transpose(cat_indices.astype(jnp.int32))
    s_a = _tc_project(t2, wtab, 0)
    s_b = _tc_project(t2, wtab, 1)
    part = _sc_gather(s_a, idxT, base, 0)
    out = _sc_gather(s_b, idxT, part, 1)
    return out.reshape(B, NOUT)
